# baseline (speedup 1.0000x reference)
import functools

import jax
import jax.numpy as jnp
from jax import lax
from jax.experimental import pallas as pl
from jax.experimental.pallas import tpu as pltpu
from jax.experimental.pallas import tpu_sc as plsc

EPS = 1e-06
K = 20
NEG_SLOPE = 0.2
TN = 256
TE = 256


def _topk_body(x_full_ref, x_tile_ref, xxc_ref, xxq_ref, idx_ref):
    xb = x_full_ref[0]
    xt = x_tile_ref[0]
    n = xb.shape[1]
    g = lax.dot_general(xb.astype(jnp.bfloat16), xt.astype(jnp.bfloat16),
                        (((0,), (0,)), ((), ())),
                        preferred_element_type=jnp.float32)
    inner = -2.0 * g
    xxc = xxc_ref[0]
    xxq = xxq_ref[0]
    pw = ((-xxc) - inner) - xxq
    vals = -pw
    iota_s32 = lax.broadcasted_iota(jnp.int32, (n, TN), 0)
    iota_s = iota_s32.astype(jnp.float32)
    big = jnp.float32(3.4e38)
    sent = jnp.float32(n * 2)
    for j in range(K):
        m = jnp.min(vals, axis=0, keepdims=True)
        cand = jnp.where(vals <= m, iota_s, sent)
        amin = jnp.min(cand, axis=0, keepdims=True)
        idx_ref[0, j, :] = amin[0].astype(jnp.int32)
        vals = jnp.where(cand == amin, big, vals)


def _topk(x):
    b, _, n = x.shape
    xx = jnp.sum(x ** 2, axis=1)
    grid = (b, n // TN)
    return pl.pallas_call(
        _topk_body,
        grid=grid,
        in_specs=[
            pl.BlockSpec((1, 3, n), lambda bi, ti: (bi, 0, 0)),
            pl.BlockSpec((1, 3, TN), lambda bi, ti: (bi, 0, ti)),
            pl.BlockSpec((1, n, 1), lambda bi, ti: (bi, 0, 0)),
            pl.BlockSpec((1, 1, TN), lambda bi, ti: (bi, 0, ti)),
        ],
        out_specs=pl.BlockSpec((1, K, TN), lambda bi, ti: (bi, 0, ti)),
        out_shape=jax.ShapeDtypeStruct((b, K, n), jnp.int32),
    )(x, x, xx.reshape(b, n, 1), xx.reshape(b, 1, n))


def _sc_gather(x, idx):
    b, _, n = x.shape
    info = plsc.get_sparse_core_info()
    nc, ns = info.num_cores, info.num_subcores
    nw = nc * ns
    npairs = b * K
    per_w = pl.cdiv(npairs, nw)
    mesh = plsc.VectorSubcoreMesh(core_axis_name="c", subcore_axis_name="s")

    @functools.partial(
        pl.kernel,
        mesh=mesh,
        compiler_params=pltpu.CompilerParams(needs_layout_passes=False),
        out_type=jax.ShapeDtypeStruct((b * 3 * K * n,), jnp.float32),
        scratch_types=[
            pltpu.VMEM((n,), jnp.float32),
            pltpu.VMEM((n,), jnp.float32),
            pltpu.VMEM((n,), jnp.float32),
            pltpu.VMEM((n,), jnp.int32),
            pltpu.VMEM((n,), jnp.float32),
        ],
    )
    def gather_kernel(x_hbm, idx_hbm, out_hbm, xb0_v, xb1_v, xb2_v,
                      idx_v, row_v):
        wid = lax.axis_index("s") * nc + lax.axis_index("c")
        xbs = [xb0_v, xb1_v, xb2_v]
        for t in range(per_w):
            pair = wid * per_w + t
            bi = pair // K
            ji = pair - bi * K
            for d in range(3):
                pltpu.sync_copy(x_hbm.at[pl.ds((bi * 3 + d) * n, n)],
                                xbs[d])
            pltpu.sync_copy(idx_hbm.at[pl.ds((bi * K + ji) * n, n)], idx_v)
            for d in range(3):

                def body(g, _, xb_v=xbs[d]):
                    i16 = idx_v[pl.ds(g * 16, 16)]
                    row_v[pl.ds(g * 16, 16)] = plsc.load_gather(
                        xb_v, [i16])
                    return _

                lax.fori_loop(0, n // 16, body, 0)
                pltpu.sync_copy(
                    row_v,
                    out_hbm.at[pl.ds(((bi * 3 + d) * K + ji) * n, n)])

    return gather_kernel(x.reshape(-1), idx.reshape(-1)).reshape(
        b, 3, K, n)


def _stats_body(x_ref, xj_ref, wf_ref, out_ref):
    bi = pl.program_id(0)
    ti = pl.program_id(1)

    @pl.when(jnp.logical_and(bi == 0, ti == 0))
    def _():
        for c in range(21):
            out_ref[0, c] = 0.0
            out_ref[1, c] = 0.0

    xi = x_ref[0]
    xj = xj_ref[0]
    e = [xj[d] - xi[d][None, :] for d in range(3)]
    ci = [jnp.broadcast_to(xi[d][None, :], (K, TE)) for d in range(3)]
    for c in range(21):
        w0 = wf_ref[c, 0]
        w1 = wf_ref[c, 1]
        nsq = jnp.zeros((K, TE), jnp.float32)
        for d in range(3):
            p = w0 * e[d] + w1 * ci[d]
            nsq = nsq + p * p
        nrm = jnp.sqrt(nsq) + EPS
        out_ref[0, c] += jnp.sum(nrm)
        out_ref[1, c] += jnp.sum(nrm * nrm)


def _stats(x, xj, w_feat):
    b, _, n = x.shape
    grid = (b, n // TE)
    return pl.pallas_call(
        _stats_body,
        grid=grid,
        in_specs=[
            pl.BlockSpec((1, 3, TE), lambda bi, ti: (bi, 0, ti)),
            pl.BlockSpec((1, 3, K, TE), lambda bi, ti: (bi, 0, 0, ti)),
            pl.BlockSpec(memory_space=pltpu.SMEM),
        ],
        out_specs=pl.BlockSpec(memory_space=pltpu.SMEM),
        out_shape=jax.ShapeDtypeStruct((2, 21), jnp.float32),
    )(x, xj, w_feat)


def _edge_body(x_ref, xj_ref, wf_ref, wd_ref, gm_ref, bt_ref, s_ref,
               out_ref):
    xi = x_ref[0]
    xj = xj_ref[0]
    cnt = s_ref[2, 0]
    e = [xj[d] - xi[d][None, :] for d in range(3)]
    ci = [jnp.broadcast_to(xi[d][None, :], (K, TE)) for d in range(3)]
    for c in range(21):
        mean = s_ref[0, c] / cnt
        var = s_ref[1, c] / cnt - mean * mean
        a = gm_ref[c] * lax.rsqrt(var + 1e-05)
        bb = bt_ref[c] - a * mean
        w0 = wf_ref[c, 0]
        w1 = wf_ref[c, 1]
        v0 = wd_ref[c, 0]
        v1 = wd_ref[c, 1]
        p = [w0 * e[dd] + w1 * ci[dd] for dd in range(3)]
        d = [v0 * e[dd] + v1 * ci[dd] for dd in range(3)]
        nsq = p[0] * p[0] + p[1] * p[1] + p[2] * p[2]
        nrm = jnp.sqrt(nsq) + EPS
        scale = (a * nrm + bb) / nrm
        ps = [scale * p[dd] for dd in range(3)]
        dot = ps[0] * d[0] + ps[1] * d[1] + ps[2] * d[2]
        dsq = d[0] * d[0] + d[1] * d[1] + d[2] * d[2]
        coef = jnp.where(dot >= 0.0, 0.0,
                         (1.0 - NEG_SLOPE) * dot / (dsq + EPS))
        for dd in range(3):
            r = ps[dd] - coef * d[dd]
            out_ref[0, c, dd, :] = jnp.mean(r, axis=0)


def _edge(x, xj, w_feat, w_dir, gamma, beta, sums):
    b, _, n = x.shape
    grid = (b, n // TE)
    return pl.pallas_call(
        _edge_body,
        grid=grid,
        in_specs=[
            pl.BlockSpec((1, 3, TE), lambda bi, ti: (bi, 0, ti)),
            pl.BlockSpec((1, 3, K, TE), lambda bi, ti: (bi, 0, 0, ti)),
            pl.BlockSpec(memory_space=pltpu.SMEM),
            pl.BlockSpec(memory_space=pltpu.SMEM),
            pl.BlockSpec(memory_space=pltpu.SMEM),
            pl.BlockSpec(memory_space=pltpu.SMEM),
            pl.BlockSpec(memory_space=pltpu.SMEM),
        ],
        out_specs=pl.BlockSpec((1, 21, 3, TE), lambda bi, ti: (bi, 0, 0, ti)),
        out_shape=jax.ShapeDtypeStruct((b, 21, 3, n), jnp.float32),
    )(x, xj, w_feat, w_dir, gamma, beta, sums)


def kernel(x, W_feat, W_dir, gamma, beta):
    b, _, n = x.shape
    idx = _topk(x)
    xj = _sc_gather(x, idx)
    sums = _stats(x, xj, W_feat)
    cnt = jnp.full((1, 21), float(b * n * K), jnp.float32)
    sums3 = jnp.concatenate([sums, cnt], axis=0)
    return _edge(x, xj, W_feat, W_dir, gamma, beta, sums3)


# speedup vs baseline: 1.0236x; 1.0236x over previous
import functools

import jax
import jax.numpy as jnp
from jax import lax
from jax.experimental import pallas as pl
from jax.experimental.pallas import tpu as pltpu
from jax.experimental.pallas import tpu_sc as plsc

EPS = 1e-06
K = 20
NEG_SLOPE = 0.2
TN = 512
TE = 256


def _topk_body(x_full_ref, x_tile_ref, xxc_ref, xxq_ref, idx_ref):
    xb = x_full_ref[0]
    xt = x_tile_ref[0]
    n = xb.shape[1]
    g = lax.dot_general(xb.astype(jnp.bfloat16), xt.astype(jnp.bfloat16),
                        (((0,), (0,)), ((), ())),
                        preferred_element_type=jnp.float32)
    inner = -2.0 * g
    xxc = xxc_ref[0]
    xxq = xxq_ref[0]
    pw = ((-xxc) - inner) - xxq
    vals = -pw
    iota_s32 = lax.broadcasted_iota(jnp.int32, (n, TN), 0)
    iota_s = iota_s32.astype(jnp.float32)
    big = jnp.float32(3.4e38)
    sent = jnp.float32(n * 2)
    for j in range(K):
        m = jnp.min(vals, axis=0, keepdims=True)
        cand = jnp.where(vals <= m, iota_s, sent)
        amin = jnp.min(cand, axis=0, keepdims=True)
        idx_ref[0, j, :] = amin[0].astype(jnp.int32)
        vals = jnp.where(cand == amin, big, vals)


def _topk(x):
    b, _, n = x.shape
    xx = jnp.sum(x ** 2, axis=1)
    grid = (b, n // TN)
    return pl.pallas_call(
        _topk_body,
        grid=grid,
        in_specs=[
            pl.BlockSpec((1, 3, n), lambda bi, ti: (bi, 0, 0)),
            pl.BlockSpec((1, 3, TN), lambda bi, ti: (bi, 0, ti)),
            pl.BlockSpec((1, n, 1), lambda bi, ti: (bi, 0, 0)),
            pl.BlockSpec((1, 1, TN), lambda bi, ti: (bi, 0, ti)),
        ],
        out_specs=pl.BlockSpec((1, K, TN), lambda bi, ti: (bi, 0, ti)),
        out_shape=jax.ShapeDtypeStruct((b, K, n), jnp.int32),
    )(x, x, xx.reshape(b, n, 1), xx.reshape(b, 1, n))


def _sc_gather(x, idx):
    b, _, n = x.shape
    info = plsc.get_sparse_core_info()
    nc, ns = info.num_cores, info.num_subcores
    nw = nc * ns
    npairs = b * K
    per_w = pl.cdiv(npairs, nw)
    mesh = plsc.VectorSubcoreMesh(core_axis_name="c", subcore_axis_name="s")

    @functools.partial(
        pl.kernel,
        mesh=mesh,
        compiler_params=pltpu.CompilerParams(needs_layout_passes=False),
        out_type=jax.ShapeDtypeStruct((b * 3 * K * n,), jnp.float32),
        scratch_types=[
            pltpu.VMEM((n,), jnp.float32),
            pltpu.VMEM((n,), jnp.float32),
            pltpu.VMEM((n,), jnp.float32),
            pltpu.VMEM((n,), jnp.int32),
            pltpu.VMEM((n,), jnp.float32),
        ],
    )
    def gather_kernel(x_hbm, idx_hbm, out_hbm, xb0_v, xb1_v, xb2_v,
                      idx_v, row_v):
        wid = lax.axis_index("s") * nc + lax.axis_index("c")
        xbs = [xb0_v, xb1_v, xb2_v]
        for t in range(per_w):
            pair = wid * per_w + t
            bi = pair // K
            ji = pair - bi * K
            for d in range(3):
                pltpu.sync_copy(x_hbm.at[pl.ds((bi * 3 + d) * n, n)],
                                xbs[d])
            pltpu.sync_copy(idx_hbm.at[pl.ds((bi * K + ji) * n, n)], idx_v)
            for d in range(3):

                def body(g, _, xb_v=xbs[d]):
                    i16 = idx_v[pl.ds(g * 16, 16)]
                    row_v[pl.ds(g * 16, 16)] = plsc.load_gather(
                        xb_v, [i16])
                    return _

                lax.fori_loop(0, n // 16, body, 0)
                pltpu.sync_copy(
                    row_v,
                    out_hbm.at[pl.ds(((bi * 3 + d) * K + ji) * n, n)])

    return gather_kernel(x.reshape(-1), idx.reshape(-1)).reshape(
        b, 3, K, n)


def _stats_body(x_ref, xj_ref, wf_ref, out_ref):
    bi = pl.program_id(0)
    ti = pl.program_id(1)

    @pl.when(jnp.logical_and(bi == 0, ti == 0))
    def _():
        for c in range(21):
            out_ref[0, c] = 0.0
            out_ref[1, c] = 0.0

    xi = x_ref[0]
    xj = xj_ref[0]
    e = [xj[d] - xi[d][None, :] for d in range(3)]
    ci = [jnp.broadcast_to(xi[d][None, :], (K, TE)) for d in range(3)]
    for c in range(21):
        w0 = wf_ref[c, 0]
        w1 = wf_ref[c, 1]
        nsq = jnp.zeros((K, TE), jnp.float32)
        for d in range(3):
            p = w0 * e[d] + w1 * ci[d]
            nsq = nsq + p * p
        nrm = jnp.sqrt(nsq) + EPS
        out_ref[0, c] += jnp.sum(nrm)
        out_ref[1, c] += jnp.sum(nrm * nrm)


def _stats(x, xj, w_feat):
    b, _, n = x.shape
    grid = (b, n // TE)
    return pl.pallas_call(
        _stats_body,
        grid=grid,
        in_specs=[
            pl.BlockSpec((1, 3, TE), lambda bi, ti: (bi, 0, ti)),
            pl.BlockSpec((1, 3, K, TE), lambda bi, ti: (bi, 0, 0, ti)),
            pl.BlockSpec(memory_space=pltpu.SMEM),
        ],
        out_specs=pl.BlockSpec(memory_space=pltpu.SMEM),
        out_shape=jax.ShapeDtypeStruct((2, 21), jnp.float32),
    )(x, xj, w_feat)


def _edge_body(x_ref, xj_ref, wf_ref, wd_ref, gm_ref, bt_ref, s_ref,
               out_ref):
    xi = x_ref[0]
    xj = xj_ref[0]
    cnt = s_ref[2, 0]
    e = [xj[d] - xi[d][None, :] for d in range(3)]
    ci = [jnp.broadcast_to(xi[d][None, :], (K, TE)) for d in range(3)]
    for c in range(21):
        mean = s_ref[0, c] / cnt
        var = s_ref[1, c] / cnt - mean * mean
        a = gm_ref[c] * lax.rsqrt(var + 1e-05)
        bb = bt_ref[c] - a * mean
        w0 = wf_ref[c, 0]
        w1 = wf_ref[c, 1]
        v0 = wd_ref[c, 0]
        v1 = wd_ref[c, 1]
        p = [w0 * e[dd] + w1 * ci[dd] for dd in range(3)]
        d = [v0 * e[dd] + v1 * ci[dd] for dd in range(3)]
        nsq = p[0] * p[0] + p[1] * p[1] + p[2] * p[2]
        nrm = jnp.sqrt(nsq) + EPS
        scale = (a * nrm + bb) / nrm
        ps = [scale * p[dd] for dd in range(3)]
        dot = ps[0] * d[0] + ps[1] * d[1] + ps[2] * d[2]
        dsq = d[0] * d[0] + d[1] * d[1] + d[2] * d[2]
        coef = jnp.where(dot >= 0.0, 0.0,
                         (1.0 - NEG_SLOPE) * dot / (dsq + EPS))
        for dd in range(3):
            r = ps[dd] - coef * d[dd]
            out_ref[0, c, dd, :] = jnp.mean(r, axis=0)


def _edge(x, xj, w_feat, w_dir, gamma, beta, sums):
    b, _, n = x.shape
    grid = (b, n // TE)
    return pl.pallas_call(
        _edge_body,
        grid=grid,
        in_specs=[
            pl.BlockSpec((1, 3, TE), lambda bi, ti: (bi, 0, ti)),
            pl.BlockSpec((1, 3, K, TE), lambda bi, ti: (bi, 0, 0, ti)),
            pl.BlockSpec(memory_space=pltpu.SMEM),
            pl.BlockSpec(memory_space=pltpu.SMEM),
            pl.BlockSpec(memory_space=pltpu.SMEM),
            pl.BlockSpec(memory_space=pltpu.SMEM),
            pl.BlockSpec(memory_space=pltpu.SMEM),
        ],
        out_specs=pl.BlockSpec((1, 21, 3, TE), lambda bi, ti: (bi, 0, 0, ti)),
        out_shape=jax.ShapeDtypeStruct((b, 21, 3, n), jnp.float32),
    )(x, xj, w_feat, w_dir, gamma, beta, sums)


def kernel(x, W_feat, W_dir, gamma, beta):
    b, _, n = x.shape
    idx = _topk(x)
    xj = _sc_gather(x, idx)
    sums = _stats(x, xj, W_feat)
    cnt = jnp.full((1, 21), float(b * n * K), jnp.float32)
    sums3 = jnp.concatenate([sums, cnt], axis=0)
    return _edge(x, xj, W_feat, W_dir, gamma, beta, sums3)
